# baseline (speedup 1.0000x reference)
import functools

import jax
import jax.numpy as jnp
from jax.experimental import pallas as pl

E = 8
H = 64
B = 2048


def _mlp_block_kernel(x_ref, w1s_ref, b1s_ref, w2s_ref, b2_ref, w3b_ref,
                      out_ref):
    xb = x_ref[...]
    cell = (jnp.clip((xb + 1.0) * 0.5, 0.0, 0.99) * 2.0).astype(jnp.int32)
    flat = (cell[:, 0:1] + 2 * cell[:, 1:2] + 4 * cell[:, 2:3])

    h1 = jnp.maximum(
        jnp.dot(xb, w1s_ref[...], preferred_element_type=jnp.float32)
        + b1s_ref[...], 0.0)

    col = jax.lax.broadcasted_iota(jnp.int32, (xb.shape[0], E * H), 1)
    a1 = jnp.where((col // H) == flat, h1, 0.0)

    col8 = jax.lax.broadcasted_iota(jnp.int32, (xb.shape[0], E), 1)
    onehot = (col8 == flat).astype(jnp.float32)

    b2sel = jnp.dot(onehot, b2_ref[...], preferred_element_type=jnp.float32)
    h2 = jnp.maximum(
        jnp.dot(a1.astype(jnp.bfloat16), w2s_ref[...],
                preferred_element_type=jnp.float32) + b2sel, 0.0)

    w3b = jnp.dot(onehot, w3b_ref[...], preferred_element_type=jnp.float32)
    y = jnp.sum(h2 * w3b[:, :H], axis=1, keepdims=True) + w3b[:, H:H + 1]
    out_ref[...] = y


@functools.partial(jax.jit, static_argnames=())
def kernel(x, W1, b1, W2, b2, W3, b3, local_min, local_max):
    n = x.shape[0]
    span = local_max - local_min
    a = 2.0 / span
    c = -1.0 - 2.0 * local_min / span
    w1p = a[:, :, None] * W1
    b1p = jnp.einsum('ed,edh->eh', c, W1) + b1
    w1s = jnp.transpose(w1p, (1, 0, 2)).reshape(3, E * H)
    b1s = b1p.reshape(1, E * H)
    w2s = W2.reshape(E * H, H).astype(jnp.bfloat16)
    w3b = jnp.concatenate([W3[:, :, 0], b3], axis=1)

    grid = (n // B,)
    out = pl.pallas_call(
        _mlp_block_kernel,
        grid=grid,
        in_specs=[
            pl.BlockSpec((B, 3), lambda i: (i, 0)),
            pl.BlockSpec((3, E * H), lambda i: (0, 0)),
            pl.BlockSpec((1, E * H), lambda i: (0, 0)),
            pl.BlockSpec((E * H, H), lambda i: (0, 0)),
            pl.BlockSpec((E, H), lambda i: (0, 0)),
            pl.BlockSpec((E, H + 1), lambda i: (0, 0)),
        ],
        out_specs=pl.BlockSpec((B, 1), lambda i: (i, 0)),
        out_shape=jax.ShapeDtypeStruct((n, 1), jnp.float32),
    )(x, w1s, b1s, w2s, b2, w3b)
    return out


# speedup vs baseline: 1.6555x; 1.6555x over previous
import functools

import jax
import jax.numpy as jnp
from jax.experimental import pallas as pl

E = 8
H = 64
B = 2048


def _mlp_block_kernel(x_ref, w1s_ref, b1s_ref, w2s_ref, b2_ref, w3b_ref,
                      ones_ref, out_ref):
    xb = x_ref[...]
    cell = (jnp.clip((xb + 1.0) * 0.5, 0.0, 0.99) * 2.0).astype(jnp.int32)
    flat = (cell[:, 0:1] + 2 * cell[:, 1:2] + 4 * cell[:, 2:3])

    xh = xb.astype(jnp.bfloat16)
    xl = (xb - xh.astype(jnp.float32)).astype(jnp.bfloat16)
    x6 = jnp.concatenate([xh, xl], axis=1)
    h1 = jnp.maximum(
        jnp.dot(x6, w1s_ref[...], preferred_element_type=jnp.float32)
        + b1s_ref[...], 0.0)

    col = jax.lax.broadcasted_iota(jnp.int32, (xb.shape[0], E * H), 1)
    a1 = jnp.where((col // H) == flat, h1, 0.0)

    col8 = jax.lax.broadcasted_iota(jnp.int32, (xb.shape[0], E), 1)
    onehot = (col8 == flat).astype(jnp.bfloat16)

    b2sel = jnp.dot(onehot, b2_ref[...], preferred_element_type=jnp.float32)
    h2 = jnp.maximum(
        jnp.dot(a1.astype(jnp.bfloat16), w2s_ref[...],
                preferred_element_type=jnp.float32) + b2sel, 0.0)

    w3b = jnp.dot(onehot, w3b_ref[...], preferred_element_type=jnp.float32)
    prod = (h2 * w3b[:, :H]).astype(jnp.bfloat16)
    y = jnp.dot(prod, ones_ref[...],
                preferred_element_type=jnp.float32) + w3b[:, H:H + 1]
    out_ref[...] = y


@functools.partial(jax.jit, static_argnames=())
def kernel(x, W1, b1, W2, b2, W3, b3, local_min, local_max):
    n = x.shape[0]
    span = local_max - local_min
    a = 2.0 / span
    c = -1.0 - 2.0 * local_min / span
    w1p = a[:, :, None] * W1
    b1p = jnp.einsum('ed,edh->eh', c, W1) + b1
    w1s = jnp.transpose(w1p, (1, 0, 2)).reshape(3, E * H)
    w1s6 = jnp.concatenate([w1s, w1s], axis=0).astype(jnp.bfloat16)
    b1s = b1p.reshape(1, E * H)
    w2s = W2.reshape(E * H, H).astype(jnp.bfloat16)
    w3b = jnp.concatenate([W3[:, :, 0], b3], axis=1)
    ones = jnp.ones((H, 1), jnp.bfloat16)

    grid = (n // B,)
    out = pl.pallas_call(
        _mlp_block_kernel,
        grid=grid,
        in_specs=[
            pl.BlockSpec((B, 3), lambda i: (i, 0)),
            pl.BlockSpec((6, E * H), lambda i: (0, 0)),
            pl.BlockSpec((1, E * H), lambda i: (0, 0)),
            pl.BlockSpec((E * H, H), lambda i: (0, 0)),
            pl.BlockSpec((E, H), lambda i: (0, 0)),
            pl.BlockSpec((E, H + 1), lambda i: (0, 0)),
            pl.BlockSpec((H, 1), lambda i: (0, 0)),
        ],
        out_specs=pl.BlockSpec((B, 1), lambda i: (i, 0)),
        out_shape=jax.ShapeDtypeStruct((n, 1), jnp.float32),
    )(x, w1s6, b1s, w2s, b2.astype(jnp.bfloat16), w3b, ones)
    return out
